# baseline (speedup 1.0000x reference)
import jax
import jax.numpy as jnp
from jax import lax
from jax.experimental import pallas as pl
from jax.experimental.pallas import tpu as pltpu

_LN_EPS = 1e-5


def _fused_kernel(x_ref, w1_ref, aff1_ref, pk_ref, sa_ref, sat_ref,
                  w2_ref, sb_ref, sbt_ref, o_ref, scr):
    bt = x_ref.shape[0]
    hi = lax.Precision.HIGHEST

    xb = x_ref[...]
    zrow = jnp.zeros((bt, 1, 64), jnp.float32)
    groups = []
    for cin in range(2):
        ec = xb[:, cin, :, 0:64]
        oc = xb[:, cin, :, 64:128]
        g0 = jnp.concatenate([zrow, oc[:, :31, :]], axis=1)
        groups += [g0, ec, oc]
    v = jnp.concatenate(groups, axis=2).astype(jnp.bfloat16)
    v2 = v.reshape(bt * 32, 384)

    b1r = pk_ref[0:1]
    ys = []
    s1 = jnp.zeros((bt, 128), jnp.float32)
    s2 = jnp.zeros((bt, 128), jnp.float32)
    for pw in range(4):
        ypw = jnp.dot(v2, w1_ref[pw], preferred_element_type=jnp.float32)
        ypw = ypw.reshape(bt, 32, 128) + b1r
        ys.append(ypw)
        s1 = s1 + jnp.sum(ypw, axis=1)
        s2 = s2 + jnp.sum(ypw * ypw, axis=1)

    stats = jnp.concatenate([s1, s2], axis=0)
    statc = jnp.dot(stats, sa_ref[...],
                    preferred_element_type=jnp.float32, precision=hi) * (1.0 / 1024.0)
    statb = jnp.dot(statc, sat_ref[...],
                    preferred_element_type=jnp.float32, precision=hi)
    mu = statb[:bt][:, None, :]
    var = jnp.maximum(statb[bt:][:, None, :] - mu * mu, 0.0)
    rs = lax.rsqrt(var + _LN_EPS)
    for pw in range(4):
        ya = (ys[pw] - mu) * rs * aff1_ref[0, pw] + aff1_ref[1, pw]
        scr[pw] = jnp.maximum(ya, 0.0).reshape(bt * 32, 128)

    zparts = []
    for pw in range(4):
        acc = jnp.dot(scr[pw, 0::8, :].astype(jnp.bfloat16), w2_ref[0],
                      preferred_element_type=jnp.float32)
        for lh in range(1, 8):
            acc = acc + jnp.dot(scr[pw, lh::8, :].astype(jnp.bfloat16),
                                w2_ref[lh], preferred_element_type=jnp.float32)
        zparts.append(acc)
    z = jnp.concatenate(zparts, axis=1) + pk_ref[1:2]

    zst = jnp.concatenate([z, z * z], axis=0)
    zc = jnp.dot(zst, sb_ref[...],
                 preferred_element_type=jnp.float32, precision=hi) * (1.0 / 32.0)
    zb = jnp.dot(zc, sbt_ref[...],
                 preferred_element_type=jnp.float32, precision=hi)
    n = bt * 4
    mu2 = zb[:n]
    var2 = jnp.maximum(zb[n:] - mu2 * mu2, 0.0)
    zo = (z - mu2) * lax.rsqrt(var2 + _LN_EPS) * pk_ref[2:3] + pk_ref[3:4]
    o_ref[...] = jnp.maximum(zo, 0.0).reshape(bt, 4, 128)


def kernel(x, w1, b1, ln1_g, ln1_b, w2, b2, ln2_g, ln2_b):
    B = x.shape[0]
    C1, Cin, KH, KW = w1.shape

    xr = x.reshape(B, x.shape[1], 32, 128)

    wcol = jnp.arange(64)[None, :, None]
    wo = jnp.arange(32)[None, None, :]
    kwi = jnp.arange(KW)[:, None, None]
    ek = (wcol == 2 * wo + kwi - 1).astype(jnp.float32)
    w1v = jnp.einsum('cikj,jwo->ikwco', w1.astype(jnp.float32), ek)
    w1v4 = (w1v.reshape(Cin, KH, 64, C1, 4, 8).transpose(4, 0, 1, 2, 3, 5)
            .reshape(4, Cin * KH * 64, C1 * 8).astype(jnp.bfloat16))

    def _aff(a):
        t = a.astype(jnp.float32).reshape(32, 4, 8).transpose(1, 0, 2)
        return jnp.tile(t[:, :, None, :], (1, 1, C1, 1)).reshape(4, 32, 128)
    aff1 = jnp.stack([_aff(ln1_g), _aff(ln1_b)])

    H = w2.shape[0]
    pk = jnp.stack([
        jnp.repeat(b1.astype(jnp.float32), 8),
        jnp.tile(b2.astype(jnp.float32), 4),
        jnp.tile(ln2_g.astype(jnp.float32), 4),
        jnp.tile(ln2_b.astype(jnp.float32), 4),
    ])

    sa = jnp.repeat(jnp.eye(C1, dtype=jnp.float32), 8, axis=0)
    sat = sa.T
    sb = jnp.repeat(jnp.eye(4, dtype=jnp.float32), H, axis=0)
    sbt = sb.T

    w2c = (w2.astype(jnp.float32).transpose(2, 1, 3, 0)
           .reshape(8, 128, H).astype(jnp.bfloat16))

    bt = 32
    while B % bt or (B // bt) < 2:
        bt //= 2
        if bt <= 1:
            bt = 1
            break

    nsteps = B // bt
    half = max(nsteps // 2, 1)
    grid = (nsteps // half, half)

    out = pl.pallas_call(
        _fused_kernel,
        out_shape=jax.ShapeDtypeStruct((B, 4, 4 * H), jnp.float32),
        grid=grid,
        in_specs=[
            pl.BlockSpec((bt, Cin, 32, 128),
                         lambda i, j: (i * half + j, 0, 0, 0)),
            pl.BlockSpec((4, 384, 128), lambda i, j: (0, 0, 0)),
            pl.BlockSpec((2, 4, 32, 128), lambda i, j: (0, 0, 0, 0)),
            pl.BlockSpec((4, 128), lambda i, j: (0, 0)),
            pl.BlockSpec((128, C1), lambda i, j: (0, 0)),
            pl.BlockSpec((C1, 128), lambda i, j: (0, 0)),
            pl.BlockSpec((8, 128, H), lambda i, j: (0, 0, 0)),
            pl.BlockSpec((128, 4), lambda i, j: (0, 0)),
            pl.BlockSpec((4, 128), lambda i, j: (0, 0)),
        ],
        out_specs=pl.BlockSpec((bt, 4, 4 * H),
                               lambda i, j: (i * half + j, 0, 0)),
        scratch_shapes=[pltpu.VMEM((4, bt * 32, 128), jnp.float32)],
        compiler_params=pltpu.CompilerParams(
            dimension_semantics=("parallel", "arbitrary"),
            vmem_limit_bytes=64 * 1024 * 1024),
    )(xr, w1v4, aff1, pk, sa, sat, w2c, sb, sbt)
    return out.reshape(B, 16, H)


# speedup vs baseline: 1.0400x; 1.0400x over previous
import jax
import jax.numpy as jnp
from jax import lax
from jax.experimental import pallas as pl
from jax.experimental.pallas import tpu as pltpu

_LN_EPS = 1e-5


def _fused_kernel(x_ref, w1m_ref, b1_ref, g1_ref, be1_ref, b2_ref, g2_ref,
                  be2_ref, sa_ref, sat_ref, w2_ref, sb_ref, sbt_ref, o_ref,
                  scr, w1s, afs, pks):
    bt = x_ref.shape[0]
    hi = lax.Precision.HIGHEST

    @pl.when(pl.program_id(1) == 0)
    def _prep():
        lsel = (lax.broadcasted_iota(jnp.int32, (16, 128), 1) // 8
                == lax.broadcasted_iota(jnp.int32, (16, 128), 0)
                ).astype(jnp.float32)
        w1cc = lax.dot_general(w1m_ref[...], lsel, (((0,), (0,)), ((), ())),
                               preferred_element_type=jnp.float32,
                               precision=hi)
        riota = lax.broadcasted_iota(jnp.int32, (384, 18), 0)
        kiota = lax.broadcasted_iota(jnp.int32, (384, 18), 1)
        rowm = lax.broadcasted_iota(jnp.int32, (384, 128), 0)
        colm = lax.broadcasted_iota(jnp.int32, (384, 128), 1)
        for pw in range(4):
            acc = jnp.zeros((384, 128), jnp.float32)
            for j in range(3):
                rk = (kiota == (riota // 64) * 3 + j).astype(jnp.float32)
                wexp = jnp.dot(rk, w1cc, preferred_element_type=jnp.float32,
                               precision=hi)
                band = rowm % 64 == 2 * (8 * pw + colm % 8) + j - 1
                acc = acc + jnp.where(band, wexp, 0.0)
            w1s[pw] = acc.astype(jnp.bfloat16)
        woi = lax.broadcasted_iota(jnp.int32, (32, 128), 0)
        coli = lax.broadcasted_iota(jnp.int32, (32, 128), 1)
        for pw in range(4):
            e = (woi == 8 * pw + coli % 8).astype(jnp.float32)
            afs[0, pw] = jnp.dot(g1_ref[...], e,
                                 preferred_element_type=jnp.float32,
                                 precision=hi)
            afs[1, pw] = jnp.dot(be1_ref[...], e,
                                 preferred_element_type=jnp.float32,
                                 precision=hi)
        hsel = (lax.broadcasted_iota(jnp.int32, (32, 128), 0)
                == lax.broadcasted_iota(jnp.int32, (32, 128), 1) % 32
                ).astype(jnp.float32)
        pks[...] = jnp.concatenate([
            jnp.dot(b1_ref[...], lsel, preferred_element_type=jnp.float32,
                    precision=hi),
            jnp.dot(b2_ref[...], hsel, preferred_element_type=jnp.float32,
                    precision=hi),
            jnp.dot(g2_ref[...], hsel, preferred_element_type=jnp.float32,
                    precision=hi),
            jnp.dot(be2_ref[...], hsel, preferred_element_type=jnp.float32,
                    precision=hi),
        ], axis=0)

    xb = x_ref[...]
    zrow = jnp.zeros((bt, 1, 64), jnp.float32)
    groups = []
    for cin in range(2):
        ec = xb[:, cin, :, 0:64]
        oc = xb[:, cin, :, 64:128]
        g0 = jnp.concatenate([zrow, oc[:, :31, :]], axis=1)
        groups += [g0, ec, oc]
    v = jnp.concatenate(groups, axis=2).astype(jnp.bfloat16)
    v2 = v.reshape(bt * 32, 384)

    b1r = pks[0:1]
    ys = []
    s1 = jnp.zeros((bt, 128), jnp.float32)
    s2 = jnp.zeros((bt, 128), jnp.float32)
    for pw in range(4):
        ypw = jnp.dot(v2, w1s[pw], preferred_element_type=jnp.float32)
        ypw = ypw.reshape(bt, 32, 128) + b1r
        ys.append(ypw)
        s1 = s1 + jnp.sum(ypw, axis=1)
        s2 = s2 + jnp.sum(ypw * ypw, axis=1)

    stats = jnp.concatenate([s1, s2], axis=0)
    statc = jnp.dot(stats, sa_ref[...],
                    preferred_element_type=jnp.float32, precision=hi) * (1.0 / 1024.0)
    statb = jnp.dot(statc, sat_ref[...],
                    preferred_element_type=jnp.float32, precision=hi)
    mu = statb[:bt][:, None, :]
    var = jnp.maximum(statb[bt:][:, None, :] - mu * mu, 0.0)
    rs = lax.rsqrt(var + _LN_EPS)
    for pw in range(4):
        ya = (ys[pw] - mu) * rs * afs[0, pw] + afs[1, pw]
        scr[pw] = jnp.maximum(ya, 0.0).reshape(bt * 32, 128)

    zparts = []
    for pw in range(4):
        acc = jnp.dot(scr[pw, 0::8, :].astype(jnp.bfloat16), w2_ref[0],
                      preferred_element_type=jnp.float32)
        for lh in range(1, 8):
            acc = acc + jnp.dot(scr[pw, lh::8, :].astype(jnp.bfloat16),
                                w2_ref[lh], preferred_element_type=jnp.float32)
        zparts.append(acc)
    z = jnp.concatenate(zparts, axis=1) + pks[1:2]

    zst = jnp.concatenate([z, z * z], axis=0)
    zc = jnp.dot(zst, sb_ref[...],
                 preferred_element_type=jnp.float32, precision=hi) * (1.0 / 32.0)
    zb = jnp.dot(zc, sbt_ref[...],
                 preferred_element_type=jnp.float32, precision=hi)
    n = bt * 4
    mu2 = zb[:n]
    var2 = jnp.maximum(zb[n:] - mu2 * mu2, 0.0)
    zo = (z - mu2) * lax.rsqrt(var2 + _LN_EPS) * pks[2:3] + pks[3:4]
    o_ref[...] = jnp.maximum(zo, 0.0).reshape(bt, 4, 128)


def kernel(x, w1, b1, ln1_g, ln1_b, w2, b2, ln2_g, ln2_b):
    B = x.shape[0]
    C1, Cin, KH, KW = w1.shape
    H = w2.shape[0]

    xr = x.reshape(B, x.shape[1], 32, 128)
    w1m = w1.astype(jnp.float32).reshape(C1, Cin * KH * KW)
    b1r0 = b1.astype(jnp.float32).reshape(1, C1)
    g1r0 = ln1_g.astype(jnp.float32)
    be1r0 = ln1_b.astype(jnp.float32)
    b2r0 = b2.astype(jnp.float32).reshape(1, H)
    g2r0 = ln2_g.astype(jnp.float32).reshape(1, H)
    be2r0 = ln2_b.astype(jnp.float32).reshape(1, H)

    sa = jnp.repeat(jnp.eye(C1, dtype=jnp.float32), 8, axis=0)
    sat = sa.T
    sb = jnp.repeat(jnp.eye(4, dtype=jnp.float32), H, axis=0)
    sbt = sb.T

    w2c = (w2.astype(jnp.float32).transpose(2, 1, 3, 0)
           .reshape(8, C1 * 8, H).astype(jnp.bfloat16))

    bt = 32
    while B % bt or (B // bt) < 2:
        bt //= 2
        if bt <= 1:
            bt = 1
            break

    nsteps = B // bt
    half = max(nsteps // 2, 1)
    grid = (nsteps // half, half)

    out = pl.pallas_call(
        _fused_kernel,
        out_shape=jax.ShapeDtypeStruct((B, 4, 4 * H), jnp.float32),
        grid=grid,
        in_specs=[
            pl.BlockSpec((bt, Cin, 32, 128),
                         lambda i, j: (i * half + j, 0, 0, 0)),
            pl.BlockSpec((C1, Cin * KH * KW), lambda i, j: (0, 0)),
            pl.BlockSpec((1, C1), lambda i, j: (0, 0)),
            pl.BlockSpec((32, 32), lambda i, j: (0, 0)),
            pl.BlockSpec((32, 32), lambda i, j: (0, 0)),
            pl.BlockSpec((1, H), lambda i, j: (0, 0)),
            pl.BlockSpec((1, H), lambda i, j: (0, 0)),
            pl.BlockSpec((1, H), lambda i, j: (0, 0)),
            pl.BlockSpec((128, C1), lambda i, j: (0, 0)),
            pl.BlockSpec((C1, 128), lambda i, j: (0, 0)),
            pl.BlockSpec((8, C1 * 8, H), lambda i, j: (0, 0, 0)),
            pl.BlockSpec((128, 4), lambda i, j: (0, 0)),
            pl.BlockSpec((4, 128), lambda i, j: (0, 0)),
        ],
        out_specs=pl.BlockSpec((bt, 4, 4 * H),
                               lambda i, j: (i * half + j, 0, 0)),
        scratch_shapes=[
            pltpu.VMEM((4, bt * 32, 128), jnp.float32),
            pltpu.VMEM((4, 384, 128), jnp.bfloat16),
            pltpu.VMEM((2, 4, 32, 128), jnp.float32),
            pltpu.VMEM((4, 128), jnp.float32),
        ],
        compiler_params=pltpu.CompilerParams(
            dimension_semantics=("parallel", "arbitrary"),
            vmem_limit_bytes=64 * 1024 * 1024),
    )(xr, w1m, b1r0, g1r0, be1r0, b2r0, g2r0, be2r0, sa, sat, w2c, sb, sbt)
    return out.reshape(B, 16, H)
